# baseline (speedup 1.0000x reference)
import functools

import jax
import jax.numpy as jnp
from jax import lax
from jax.experimental import pallas as pl
from jax.experimental.pallas import tpu as pltpu
from jax.experimental.pallas import tpu_sc as plsc

N = 320000
D = 128
B = 64
NC = 2
NS = 16
NW = NC * NS
ROWS_PER_W = N // NW
R = 80
NB = 400
K = NB // R
BLOCKS = ROWS_PER_W // NB
ITERS = ROWS_PER_W // R
CW = 128
RPT = B // NS


def _sc_segment_sums(features, point_idx):
    mesh = plsc.VectorSubcoreMesh(
        core_axis_name="c", subcore_axis_name="s",
        num_cores=NC, num_subcores=NS)

    idx3 = point_idx.reshape(NW, ITERS, R)

    @functools.partial(
        pl.kernel,
        out_type=(
            jax.ShapeDtypeStruct((NC, B, D), jnp.float32),
            jax.ShapeDtypeStruct((NC, B, CW), jnp.float32),
        ),
        mesh=mesh,
        scratch_types=[
            pltpu.VMEM((ITERS, R), jnp.int32),
            pltpu.VMEM((2, NB, D), jnp.float32),
            pltpu.VMEM((R, CW), jnp.float32),
            pltpu.VMEM((RPT, CW), jnp.float32),
            pltpu.VMEM_SHARED((B, D), jnp.float32),
            pltpu.VMEM_SHARED((B, CW), jnp.float32),
            pltpu.SemaphoreType.DMA,
            pltpu.SemaphoreType.DMA,
            pltpu.SemaphoreType.DMA,
            pltpu.SemaphoreType.DMA,
            pltpu.SemaphoreType.DMA,
        ],
    )
    def seg_sum(feat_hbm, idx_hbm, sums_hbm, counts_hbm,
                idx_v, rows_v, ones_v, zc_v, acc_s, cnt_s,
                sem0, sem1, semsc0, semsc1, semcnt):
        cid = lax.axis_index("c")
        sid = lax.axis_index("s")
        wid = cid * NS + sid
        base = wid * ROWS_PER_W

        ones16 = jnp.ones((16,), jnp.float32)
        zeros16 = jnp.zeros((16,), jnp.float32)

        idx_copy = pltpu.async_copy(idx_hbm.at[wid], idx_v, sem0)

        def init_ones(i, _):
            def col(j, _):
                ones_v[i, pl.ds(16 * j, 16)] = ones16
                return 0
            lax.fori_loop(0, CW // 16, col, 0)
            return 0
        lax.fori_loop(0, R, init_ones, 0)

        def init_zrow(i, _):
            def init_zcol(j, _):
                zc_v[i, pl.ds(16 * j, 16)] = zeros16
                rows_v[0, i, pl.ds(16 * j, 16)] = zeros16
                return 0
            lax.fori_loop(0, D // 16, init_zcol, 0)
            return 0
        lax.fori_loop(0, RPT, init_zrow, 0)

        pltpu.sync_copy(rows_v.at[0, pl.ds(0, RPT), :],
                        acc_s.at[pl.ds(RPT * sid, RPT), :])
        pltpu.sync_copy(zc_v, cnt_s.at[pl.ds(RPT * sid, RPT), :])
        idx_copy.wait()
        plsc.subcore_barrier()

        def feat_copy(bi, buf):
            return pltpu.async_copy(
                feat_hbm.at[pl.ds(base + bi * NB, NB), :],
                rows_v.at[buf], sem0 if buf == 0 else sem1)

        feat_copy(0, 0)
        feat_copy(1, 1)

        def scat_wait(bi, buf):
            for j in range(K):
                pltpu.make_async_copy(
                    rows_v.at[buf, pl.ds(j * R, R), :],
                    acc_s.at[idx_v.at[bi * K + j]],
                    semsc0 if buf == 0 else semsc1).wait()

        def step(bi, _):
            def do(buf):
                pltpu.make_async_copy(
                    feat_hbm.at[pl.ds(base + bi * NB, NB), :],
                    rows_v.at[buf], sem0 if buf == 0 else sem1).wait()
                for j in range(K):
                    pltpu.async_copy(
                        rows_v.at[buf, pl.ds(j * R, R), :],
                        acc_s.at[idx_v.at[bi * K + j]],
                        semsc0 if buf == 0 else semsc1, add=True)
                    pltpu.async_copy(
                        ones_v, cnt_s.at[idx_v.at[bi * K + j]], semcnt,
                        add=True)

                @pl.when(bi >= 1)
                def _():
                    scat_wait(bi - 1, 1 - buf)

                    @pl.when(bi + 1 < BLOCKS)
                    def _():
                        feat_copy(bi + 1, 1 - buf)

            @pl.when(lax.rem(bi, 2) == 0)
            def _():
                do(0)

            @pl.when(lax.rem(bi, 2) == 1)
            def _():
                do(1)
            return 0
        lax.fori_loop(0, BLOCKS, step, 0)

        scat_wait(BLOCKS - 1, (BLOCKS - 1) % 2)

        def drain(i, _):
            pltpu.make_async_copy(ones_v, cnt_s.at[idx_v.at[0]],
                                  semcnt).wait()
            return 0
        lax.fori_loop(0, ITERS, drain, 0)

        plsc.subcore_barrier()

        @pl.when(sid == 0)
        def _():
            pltpu.sync_copy(acc_s, sums_hbm.at[cid])
            pltpu.sync_copy(cnt_s, counts_hbm.at[cid])

    return seg_sum(features, idx3)


def _tc_combine(sums, counts):
    def body(s_ref, c_ref, o_ref):
        s = s_ref[0] + s_ref[1]
        c = c_ref[0] + c_ref[1]
        o_ref[...] = s / c[:, 0:1]

    return pl.pallas_call(
        body,
        out_shape=jax.ShapeDtypeStruct((B, D), jnp.float32),
    )(sums, counts)


def kernel(features, point_idx):
    sums, counts = _sc_segment_sums(features, point_idx)
    return _tc_combine(sums, counts)


# speedup vs baseline: 1.4907x; 1.4907x over previous
import functools

import jax
import jax.numpy as jnp
from jax import lax
from jax.experimental import pallas as pl
from jax.experimental.pallas import tpu as pltpu
from jax.experimental.pallas import tpu_sc as plsc

N = 320000
D = 128
B = 64
NC = 2
NS = 16
NW = NC * NS
ROWS_PER_W = N // NW
R = 80
NB = 400
K = NB // R
BLOCKS = ROWS_PER_W // NB
ITERS = ROWS_PER_W // R
RPT = B // NS


def _sc_segment_sums(features, point_idx):
    mesh = plsc.VectorSubcoreMesh(
        core_axis_name="c", subcore_axis_name="s",
        num_cores=NC, num_subcores=NS)

    idx3 = point_idx.reshape(NW, ITERS, R)

    @functools.partial(
        pl.kernel,
        out_type=jax.ShapeDtypeStruct((NC, B, D), jnp.float32),
        mesh=mesh,
        scratch_types=[
            pltpu.VMEM((ITERS, R), jnp.int32),
            pltpu.VMEM((2, NB, D), jnp.float32),
            pltpu.VMEM_SHARED((B, D), jnp.float32),
            pltpu.SemaphoreType.DMA,
            pltpu.SemaphoreType.DMA,
            pltpu.SemaphoreType.DMA,
            pltpu.SemaphoreType.DMA,
        ],
    )
    def seg_sum(feat_hbm, idx_hbm, sums_hbm,
                idx_v, rows_v, acc_s, sem0, sem1, semsc0, semsc1):
        cid = lax.axis_index("c")
        sid = lax.axis_index("s")
        wid = cid * NS + sid
        base = wid * ROWS_PER_W

        zeros16 = jnp.zeros((16,), jnp.float32)

        idx_copy = pltpu.async_copy(idx_hbm.at[wid], idx_v, sem0)

        def init_zrow(i, _):
            def init_zcol(j, _):
                rows_v[0, i, pl.ds(16 * j, 16)] = zeros16
                return 0
            lax.fori_loop(0, D // 16, init_zcol, 0)
            return 0
        lax.fori_loop(0, RPT, init_zrow, 0)

        pltpu.sync_copy(rows_v.at[0, pl.ds(0, RPT), :],
                        acc_s.at[pl.ds(RPT * sid, RPT), :])
        idx_copy.wait()
        plsc.subcore_barrier()

        def feat_copy(bi, buf):
            return pltpu.async_copy(
                feat_hbm.at[pl.ds(base + bi * NB, NB), :],
                rows_v.at[buf], sem0 if buf == 0 else sem1)

        feat_copy(0, 0)
        feat_copy(1, 1)

        def scat_wait(bi, buf):
            for j in range(K):
                pltpu.make_async_copy(
                    rows_v.at[buf, pl.ds(j * R, R), :],
                    acc_s.at[idx_v.at[bi * K + j]],
                    semsc0 if buf == 0 else semsc1).wait()

        def step(bi, _):
            def do(buf):
                pltpu.make_async_copy(
                    feat_hbm.at[pl.ds(base + bi * NB, NB), :],
                    rows_v.at[buf], sem0 if buf == 0 else sem1).wait()
                for j in range(K):
                    pltpu.async_copy(
                        rows_v.at[buf, pl.ds(j * R, R), :],
                        acc_s.at[idx_v.at[bi * K + j]],
                        semsc0 if buf == 0 else semsc1, add=True)

                @pl.when(bi >= 1)
                def _():
                    scat_wait(bi - 1, 1 - buf)

                    @pl.when(bi + 1 < BLOCKS)
                    def _():
                        feat_copy(bi + 1, 1 - buf)

            @pl.when(lax.rem(bi, 2) == 0)
            def _():
                do(0)

            @pl.when(lax.rem(bi, 2) == 1)
            def _():
                do(1)
            return 0
        lax.fori_loop(0, BLOCKS, step, 0)

        scat_wait(BLOCKS - 1, (BLOCKS - 1) % 2)

        plsc.subcore_barrier()

        @pl.when(sid == 0)
        def _():
            pltpu.sync_copy(acc_s, sums_hbm.at[cid])

    return seg_sum(features, idx3)


def _tc_counts(point_idx):
    idx2 = point_idx.reshape(N // D, D)

    def body(i_ref, c_ref):
        idx = i_ref[...]
        for b in range(B):
            cnt = jnp.sum((idx == b).astype(jnp.float32))
            c_ref[b, :] = jnp.full((D,), cnt, jnp.float32)

    return pl.pallas_call(
        body,
        out_shape=jax.ShapeDtypeStruct((B, D), jnp.float32),
    )(idx2)


def _tc_combine(sums, counts):
    def body(s_ref, c_ref, o_ref):
        s = s_ref[0] + s_ref[1]
        o_ref[...] = s / c_ref[...]

    return pl.pallas_call(
        body,
        out_shape=jax.ShapeDtypeStruct((B, D), jnp.float32),
    )(sums, counts)


def kernel(features, point_idx):
    counts = _tc_counts(point_idx)
    sums = _sc_segment_sums(features, point_idx)
    return _tc_combine(sums, counts)
